# baseline (device time: 33595 ns/iter reference)
import jax
import jax.numpy as jnp
from jax import lax
from jax.experimental import pallas as pl
from jax.experimental.pallas import tpu as pltpu

M = 1024
D = 1024


def kernel(partial, gamma):
    def body(p_ref, g_ref, o_ref, send_buf, recv_buf, send_sem, recv_sem):
        my_x = lax.axis_index("x")
        my_y = lax.axis_index("y")
        my_z = lax.axis_index("z")
        peer = (my_x, my_y, 1 - my_z)

        barrier_sem = pltpu.get_barrier_semaphore()
        pl.semaphore_signal(
            barrier_sem, inc=1, device_id=peer,
            device_id_type=pl.DeviceIdType.MESH,
        )
        pl.semaphore_wait(barrier_sem, 1)

        peer_off = (1 - my_z) * M
        send_buf[...] = p_ref[0, pl.ds(peer_off, M), :].astype(jnp.bfloat16)
        rdma = pltpu.make_async_remote_copy(
            src_ref=send_buf,
            dst_ref=recv_buf,
            send_sem=send_sem,
            recv_sem=recv_sem,
            device_id=peer,
            device_id_type=pl.DeviceIdType.MESH,
        )
        rdma.start()
        rdma.wait()

        my_off = my_z * M
        y = p_ref[0, pl.ds(my_off, M), :] + recv_buf[...].astype(jnp.float32)
        ms = jnp.mean(y * y, axis=-1, keepdims=True)
        o_ref[...] = y * lax.rsqrt(ms + 1e-6) * g_ref[0, :]

    gamma2d = gamma.reshape(1, D)
    return pl.pallas_call(
        body,
        out_shape=jax.ShapeDtypeStruct((M, D), jnp.float32),
        in_specs=[
            pl.BlockSpec(memory_space=pltpu.VMEM),
            pl.BlockSpec(memory_space=pltpu.VMEM),
        ],
        out_specs=pl.BlockSpec(memory_space=pltpu.VMEM),
        scratch_shapes=[
            pltpu.VMEM((M, D), jnp.bfloat16),
            pltpu.VMEM((M, D), jnp.bfloat16),
            pltpu.SemaphoreType.DMA,
            pltpu.SemaphoreType.DMA,
        ],
        compiler_params=pltpu.CompilerParams(collective_id=0),
    )(partial, gamma2d)


# device time: 25170 ns/iter; 1.3347x vs baseline; 1.3347x over previous
import jax
import jax.numpy as jnp
from jax import lax
from jax.experimental import pallas as pl
from jax.experimental.pallas import tpu as pltpu

M = 1024
D = 1024
Q = 256


def kernel(partial, gamma):
    def body(
        p_ref, g_ref, o_ref,
        sendz, recv_own, recv_diag, t_own, recv_tx, recv_ty,
        sendz_sem, recvz_sem, sendp_sem, recv_tx_sem, recv_ty_sem,
    ):
        my_x = lax.axis_index("x")
        my_y = lax.axis_index("y")
        my_z = lax.axis_index("z")
        z_peer = (my_x, my_y, 1 - my_z)
        x_nb = (1 - my_x, my_y, my_z)
        y_nb = (my_x, 1 - my_y, my_z)

        qown = 2 * my_x + my_y
        qdiag = 3 - qown
        qx = 2 * (1 - my_x) + my_y
        qy = 2 * my_x + (1 - my_y)

        my_block = my_z * M
        peer_block = (1 - my_z) * M

        barrier_sem = pltpu.get_barrier_semaphore()
        for nbr in (z_peer, x_nb, y_nb):
            pl.semaphore_signal(
                barrier_sem, inc=1, device_id=nbr,
                device_id_type=pl.DeviceIdType.MESH,
            )
        pl.semaphore_wait(barrier_sem, 3)

        sendz[0] = p_ref[0, pl.ds(peer_block + qown * Q, Q), :].astype(
            jnp.bfloat16
        )
        rdma_z0 = pltpu.make_async_remote_copy(
            src_ref=sendz.at[0], dst_ref=recv_own,
            send_sem=sendz_sem.at[0], recv_sem=recvz_sem.at[0],
            device_id=z_peer, device_id_type=pl.DeviceIdType.MESH,
        )
        rdma_z0.start()
        sendz[1] = p_ref[0, pl.ds(peer_block + qdiag * Q, Q), :].astype(
            jnp.bfloat16
        )
        rdma_z1 = pltpu.make_async_remote_copy(
            src_ref=sendz.at[1], dst_ref=recv_diag,
            send_sem=sendz_sem.at[1], recv_sem=recvz_sem.at[1],
            device_id=z_peer, device_id_type=pl.DeviceIdType.MESH,
        )
        rdma_z1.start()

        g = g_ref[0, :]

        rdma_z0.wait_recv()
        y_own = (
            p_ref[0, pl.ds(my_block + qown * Q, Q), :]
            + recv_own[...].astype(jnp.float32)
        )
        ms = jnp.mean(y_own * y_own, axis=-1, keepdims=True)
        out_own = y_own * lax.rsqrt(ms + 1e-6) * g
        t_own[...] = out_own.astype(jnp.bfloat16)

        rdma_px = pltpu.make_async_remote_copy(
            src_ref=t_own, dst_ref=recv_tx,
            send_sem=sendp_sem.at[0], recv_sem=recv_tx_sem,
            device_id=x_nb, device_id_type=pl.DeviceIdType.MESH,
        )
        rdma_px.start()
        rdma_py = pltpu.make_async_remote_copy(
            src_ref=t_own, dst_ref=recv_ty,
            send_sem=sendp_sem.at[1], recv_sem=recv_ty_sem,
            device_id=y_nb, device_id_type=pl.DeviceIdType.MESH,
        )
        rdma_py.start()

        o_ref[pl.ds(qown * Q, Q), :] = out_own

        rdma_z1.wait_recv()
        y_diag = (
            p_ref[0, pl.ds(my_block + qdiag * Q, Q), :]
            + recv_diag[...].astype(jnp.float32)
        )
        ms_d = jnp.mean(y_diag * y_diag, axis=-1, keepdims=True)
        o_ref[pl.ds(qdiag * Q, Q), :] = y_diag * lax.rsqrt(ms_d + 1e-6) * g

        rdma_px.wait_recv()
        o_ref[pl.ds(qx * Q, Q), :] = recv_tx[...].astype(jnp.float32)
        rdma_py.wait_recv()
        o_ref[pl.ds(qy * Q, Q), :] = recv_ty[...].astype(jnp.float32)

        rdma_z0.wait_send()
        rdma_z1.wait_send()
        rdma_px.wait_send()
        rdma_py.wait_send()

    gamma2d = gamma.reshape(1, D)
    return pl.pallas_call(
        body,
        out_shape=jax.ShapeDtypeStruct((M, D), jnp.float32),
        in_specs=[
            pl.BlockSpec(memory_space=pltpu.VMEM),
            pl.BlockSpec(memory_space=pltpu.VMEM),
        ],
        out_specs=pl.BlockSpec(memory_space=pltpu.VMEM),
        scratch_shapes=[
            pltpu.VMEM((2, Q, D), jnp.bfloat16),
            pltpu.VMEM((Q, D), jnp.bfloat16),
            pltpu.VMEM((Q, D), jnp.bfloat16),
            pltpu.VMEM((Q, D), jnp.bfloat16),
            pltpu.VMEM((Q, D), jnp.bfloat16),
            pltpu.VMEM((Q, D), jnp.bfloat16),
            pltpu.SemaphoreType.DMA((2,)),
            pltpu.SemaphoreType.DMA((2,)),
            pltpu.SemaphoreType.DMA((2,)),
            pltpu.SemaphoreType.DMA,
            pltpu.SemaphoreType.DMA,
        ],
        compiler_params=pltpu.CompilerParams(collective_id=0),
    )(partial, gamma2d)
